# baseline (speedup 1.0000x reference)
import functools

import jax
import jax.numpy as jnp
from jax import lax
from jax.experimental import pallas as pl
from jax.experimental.pallas import tpu as pltpu
from jax.experimental.pallas import tpu_sc as plsc

VOCAB = 1_000_000
D = 32
B = 4096
H = 200

NC = 2
NS = 16
NW = NC * NS
BPW = B // NW
IPW = BPW * H
CROWS = 4
CIDX = CROWS * H
NCH = BPW // CROWS
SLICES = [(o, min(128, CIDX - o)) for o in range(0, CIDX, 128)]


def _copies(table_hbm, idx_v, buf, sem, cbase):
    out = []
    for (o, n) in SLICES:
        src = table_hbm.at[idx_v.at[pl.ds(cbase + o, n)]]
        dst = buf.at[pl.ds(o, n)]
        out.append(pltpu.make_async_copy(src, dst, sem))
    return out


def _issue(table_hbm, idx_v, buf, sem, cbase):
    for c in _copies(table_hbm, idx_v, buf, sem, cbase):
        c.start()


def _drain(table_hbm, idx_v, buf, sem, cbase):
    for c in _copies(table_hbm, idx_v, buf, sem, cbase):
        c.wait()


def _accum_row(buf, rbase):
    def body(j, accs):
        a0, a1 = accs
        a0 = a0 + buf[rbase + j, pl.ds(0, 16)]
        a1 = a1 + buf[rbase + j, pl.ds(16, 16)]
        return (a0, a1)

    z = jnp.zeros((16,), jnp.float32)
    return lax.fori_loop(0, H, body, (z, z), unroll=8)


_GATHER_DNUMS = lax.GatherDimensionNumbers(
    offset_dims=(), collapsed_slice_dims=(0,), start_index_map=(0,))


def _perm16(x, perm):
    return lax.gather(x, perm[:, None], _GATHER_DNUMS, (1,),
                      mode=lax.GatherScatterMode.PROMISE_IN_BOUNDS)


def _count_nnz(idx_v, ibase):
    lane = lax.iota(jnp.int32, 16)
    cnt = jnp.zeros((16,), jnp.int32)
    one = jnp.ones((16,), jnp.int32)
    zero = jnp.zeros((16,), jnp.int32)
    for k in range(H // 16 + 1):
        v = idx_v[pl.ds(ibase + 16 * k, 16)]
        ok = v != 0
        if k == H // 16:
            ok = jnp.logical_and(ok, lane < H - 16 * k)
        cnt = cnt + jnp.where(ok, one, zero)
    for s in (1, 2, 4, 8):
        cnt = cnt + _perm16(cnt, lane ^ s)
    return cnt


def _body(idx_hbm, table_hbm, out_hbm, idx_v, buf0, buf1, out_v, t0_v,
          sem0, sem1):
    wid = lax.axis_index("s") * NC + lax.axis_index("c")
    pltpu.sync_copy(idx_hbm.at[wid], idx_v.at[pl.ds(0, IPW)])
    pltpu.sync_copy(table_hbm.at[pl.ds(0, 1)], t0_v)
    t0a = t0_v[0, pl.ds(0, 16)]
    t0b = t0_v[0, pl.ds(16, 16)]
    bufs = (buf0, buf1)
    sems = (sem0, sem1)

    for b in (0, 1):
        _issue(table_hbm, idx_v, bufs[b], sems[b],
               pl.multiple_of(b * CIDX, 8))

    def outer(g, carry):
        for b in (0, 1):
            cc = g * 2 + b
            buf, sem = bufs[b], sems[b]
            cbase = pl.multiple_of(cc * CIDX, 8)
            _drain(table_hbm, idx_v, buf, sem, cbase)
            for r in range(CROWS):
                row = cc * CROWS + r
                a0, a1 = _accum_row(buf, r * H)
                nnz = _count_nnz(idx_v, pl.multiple_of(row * H, 8))
                nnzf = nnz.astype(jnp.float32)
                n0f = jnp.float32(H) - nnzf
                inv = 1.0 / nnzf
                out_v[row, pl.ds(0, 16)] = (a0 - n0f * t0a) * inv
                out_v[row, pl.ds(16, 16)] = (a1 - n0f * t0b) * inv
            nxt = cc + 2

            @pl.when(nxt < NCH)
            def _():
                _issue(table_hbm, idx_v, buf, sem,
                       pl.multiple_of(nxt * CIDX, 8))
        return carry

    lax.fori_loop(0, NCH // 2, outer, 0)
    base = pl.multiple_of(wid * BPW, 8)
    pltpu.sync_copy(out_v, out_hbm.at[pl.ds(base, BPW)])


_doc2vec_sc = functools.partial(
    pl.kernel,
    mesh=plsc.VectorSubcoreMesh(core_axis_name="c", subcore_axis_name="s"),
    compiler_params=pltpu.CompilerParams(use_tc_tiling_on_sc=False),
    out_type=jax.ShapeDtypeStruct((B, D), jnp.float32),
    scratch_types=[
        pltpu.VMEM((IPW + 16,), jnp.int32),
        pltpu.VMEM((CIDX, D), jnp.float32),
        pltpu.VMEM((CIDX, D), jnp.float32),
        pltpu.VMEM((BPW, D), jnp.float32),
        pltpu.VMEM((1, D), jnp.float32),
        pltpu.SemaphoreType.DMA,
        pltpu.SemaphoreType.DMA,
    ],
)(_body)


_TCHUNK = 4096
_TGRID = -(-VOCAB // _TCHUNK)


def _transpose_body(src_ref, dst_ref):
    row = lax.broadcasted_iota(jnp.int32, (D, D), 0)
    col = lax.broadcasted_iota(jnp.int32, (D, D), 1)
    eye = (row == col).astype(jnp.float32)
    dst_ref[...] = lax.dot_general(
        src_ref[...], eye, (((0,), (0,)), ((), ())),
        precision=lax.Precision.HIGHEST,
        preferred_element_type=jnp.float32)


_transpose_tc = pl.pallas_call(
    _transpose_body,
    grid=(_TGRID,),
    in_specs=[pl.BlockSpec((D, _TCHUNK), lambda i: (0, i))],
    out_specs=pl.BlockSpec((_TCHUNK, D), lambda i: (i, 0)),
    out_shape=jax.ShapeDtypeStruct((VOCAB, D), jnp.float32),
)


@jax.jit
def kernel(word_indices, table):
    idx = word_indices.reshape(NW, IPW)
    table_rm = _transpose_tc(jnp.swapaxes(table, 0, 1))
    return _doc2vec_sc(idx, table_rm)


# speedup vs baseline: 2.3531x; 2.3531x over previous
import functools

import jax
import jax.numpy as jnp
from jax import lax
from jax.experimental import pallas as pl
from jax.experimental.pallas import tpu as pltpu
from jax.experimental.pallas import tpu_sc as plsc

VOCAB = 1_000_000
D = 32
B = 4096
H = 200

NC = 2
NS = 16
NW = NC * NS
BPW = B // NW
IPW = BPW * H
CROWS = 4
CIDX = CROWS * H
NCH = BPW // CROWS
SLICES = [(o, min(128, CIDX - o)) for o in range(0, CIDX, 128)]


def _copies(table_hbm, idx_v, buf, sem, cbase):
    out = []
    for (o, n) in SLICES:
        src = table_hbm.at[idx_v.at[pl.ds(cbase + o, n)]]
        dst = buf.at[pl.ds(o, n)]
        out.append(pltpu.make_async_copy(src, dst, sem))
    return out


def _issue(table_hbm, idx_v, buf, sem, cbase):
    for c in _copies(table_hbm, idx_v, buf, sem, cbase):
        c.start()


def _drain(table_hbm, idx_v, buf, sem, cbase):
    for c in _copies(table_hbm, idx_v, buf, sem, cbase):
        c.wait()


def _accum_row(buf, rbase):
    def body(j, accs):
        a0, a1 = accs
        a0 = a0 + buf[rbase + j, pl.ds(0, 16)]
        a1 = a1 + buf[rbase + j, pl.ds(16, 16)]
        return (a0, a1)

    z = jnp.zeros((16,), jnp.float32)
    return lax.fori_loop(0, H, body, (z, z), unroll=8)


_GATHER_DNUMS = lax.GatherDimensionNumbers(
    offset_dims=(), collapsed_slice_dims=(0,), start_index_map=(0,))


def _perm16(x, perm):
    return lax.gather(x, perm[:, None], _GATHER_DNUMS, (1,),
                      mode=lax.GatherScatterMode.PROMISE_IN_BOUNDS)


def _count_nnz(idx_v, ibase):
    lane = lax.iota(jnp.int32, 16)
    cnt = jnp.zeros((16,), jnp.int32)
    one = jnp.ones((16,), jnp.int32)
    zero = jnp.zeros((16,), jnp.int32)
    for k in range(H // 16 + 1):
        v = idx_v[pl.ds(ibase + 16 * k, 16)]
        ok = v != 0
        if k == H // 16:
            ok = jnp.logical_and(ok, lane < H - 16 * k)
        cnt = cnt + jnp.where(ok, one, zero)
    for s in (1, 2, 4, 8):
        cnt = cnt + _perm16(cnt, lane ^ s)
    return cnt


def _body(idx_hbm, table_hbm, out_hbm, idx_v, buf0, buf1, out_v, t0_v,
          sem0, sem1):
    wid = lax.axis_index("s") * NC + lax.axis_index("c")
    pltpu.sync_copy(idx_hbm.at[wid], idx_v.at[pl.ds(0, IPW)])

    def _remap(k, carry):
        o = pl.multiple_of(k * 16, 8)
        v = idx_v[pl.ds(o, 16)]
        idx_v[pl.ds(o, 16)] = ((v & (-4096)) + ((v & 1023) << 2)
                               + ((v >> 10) & 3))
        return carry

    lax.fori_loop(0, IPW // 16, _remap, 0, unroll=8)
    pltpu.sync_copy(table_hbm.at[pl.ds(0, 1)], t0_v)
    t0a = t0_v[0, pl.ds(0, 16)]
    t0b = t0_v[0, pl.ds(16, 16)]
    bufs = (buf0, buf1)
    sems = (sem0, sem1)

    for b in (0, 1):
        _issue(table_hbm, idx_v, bufs[b], sems[b],
               pl.multiple_of(b * CIDX, 8))

    def outer(g, carry):
        for b in (0, 1):
            cc = g * 2 + b
            buf, sem = bufs[b], sems[b]
            cbase = pl.multiple_of(cc * CIDX, 8)
            _drain(table_hbm, idx_v, buf, sem, cbase)
            for r in range(CROWS):
                row = cc * CROWS + r
                a0, a1 = _accum_row(buf, r * H)
                nnz = _count_nnz(idx_v, pl.multiple_of(row * H, 8))
                nnzf = nnz.astype(jnp.float32)
                n0f = jnp.float32(H) - nnzf
                inv = 1.0 / nnzf
                out_v[row, pl.ds(0, 16)] = (a0 - n0f * t0a) * inv
                out_v[row, pl.ds(16, 16)] = (a1 - n0f * t0b) * inv
            nxt = cc + 2

            @pl.when(nxt < NCH)
            def _():
                _issue(table_hbm, idx_v, buf, sem,
                       pl.multiple_of(nxt * CIDX, 8))
        return carry

    lax.fori_loop(0, NCH // 2, outer, 0)
    base = pl.multiple_of(wid * BPW, 8)
    pltpu.sync_copy(out_v, out_hbm.at[pl.ds(base, BPW)])


_doc2vec_sc = functools.partial(
    pl.kernel,
    mesh=plsc.VectorSubcoreMesh(core_axis_name="c", subcore_axis_name="s"),
    compiler_params=pltpu.CompilerParams(use_tc_tiling_on_sc=False),
    out_type=jax.ShapeDtypeStruct((B, D), jnp.float32),
    scratch_types=[
        pltpu.VMEM((IPW + 16,), jnp.int32),
        pltpu.VMEM((CIDX, D), jnp.float32),
        pltpu.VMEM((CIDX, D), jnp.float32),
        pltpu.VMEM((BPW, D), jnp.float32),
        pltpu.VMEM((1, D), jnp.float32),
        pltpu.SemaphoreType.DMA,
        pltpu.SemaphoreType.DMA,
    ],
)(_body)


_TCHUNK = 1024
_TQ = 128 // D
_TGRID = -(-VOCAB // (_TCHUNK * _TQ))
_TROWS = _TGRID * _TCHUNK


def _transpose_body(src_ref, dst_ref):
    s = src_ref[...]
    dst_ref[...] = jnp.concatenate(
        [s[:, _TCHUNK * q:_TCHUNK * (q + 1)].T for q in range(_TQ)], axis=1)


_transpose_tc = pl.pallas_call(
    _transpose_body,
    grid=(_TGRID,),
    in_specs=[pl.BlockSpec((D, _TCHUNK * _TQ), lambda i: (0, i))],
    out_specs=pl.BlockSpec((_TCHUNK, 128), lambda i: (i, 0)),
    out_shape=jax.ShapeDtypeStruct((_TROWS, 128), jnp.float32),
)


@jax.jit
def kernel(word_indices, table):
    idx = word_indices.reshape(NW, IPW)
    table_pk = _transpose_tc(jnp.swapaxes(table, 0, 1))
    return _doc2vec_sc(idx, table_pk.reshape(_TROWS * _TQ, D))


# speedup vs baseline: 3.1969x; 1.3586x over previous
import functools

import jax
import jax.numpy as jnp
from jax import lax
from jax.experimental import pallas as pl
from jax.experimental.pallas import tpu as pltpu
from jax.experimental.pallas import tpu_sc as plsc

VOCAB = 1_000_000
D = 32
B = 4096
H = 200

NC = 2
NS = 16
NW = NC * NS
BPW = B // NW
IPW = BPW * H
CROWS = 4
CIDX = CROWS * H
NCH = BPW // CROWS
SLICES = [(o, min(128, CIDX - o)) for o in range(0, CIDX, 128)]


def _copies(table_hbm, idx_v, buf, sem, cbase):
    out = []
    for (o, n) in SLICES:
        src = table_hbm.at[idx_v.at[pl.ds(cbase + o, n)]]
        dst = buf.at[pl.ds(o, n)]
        out.append(pltpu.make_async_copy(src, dst, sem))
    return out


def _issue(table_hbm, idx_v, buf, sem, cbase):
    for c in _copies(table_hbm, idx_v, buf, sem, cbase):
        c.start()


def _drain(table_hbm, idx_v, buf, sem, cbase):
    for c in _copies(table_hbm, idx_v, buf, sem, cbase):
        c.wait()


def _accum_row(buf, rbase):
    def body(j, accs):
        a0, a1 = accs
        a0 = a0 + buf[rbase + j, pl.ds(0, 16)]
        a1 = a1 + buf[rbase + j, pl.ds(16, 16)]
        return (a0, a1)

    z = jnp.zeros((16,), jnp.float32)
    return lax.fori_loop(0, H, body, (z, z), unroll=8)


_GATHER_DNUMS = lax.GatherDimensionNumbers(
    offset_dims=(), collapsed_slice_dims=(0,), start_index_map=(0,))


def _perm16(x, perm):
    return lax.gather(x, perm[:, None], _GATHER_DNUMS, (1,),
                      mode=lax.GatherScatterMode.PROMISE_IN_BOUNDS)


def _count_nnz(idx_v, ibase):
    lane = lax.iota(jnp.int32, 16)
    cnt = jnp.zeros((16,), jnp.int32)
    one = jnp.ones((16,), jnp.int32)
    zero = jnp.zeros((16,), jnp.int32)
    for k in range(H // 16 + 1):
        v = idx_v[pl.ds(ibase + 16 * k, 16)]
        ok = v != 0
        if k == H // 16:
            ok = jnp.logical_and(ok, lane < H - 16 * k)
        cnt = cnt + jnp.where(ok, one, zero)
    for s in (1, 2, 4, 8):
        cnt = cnt + _perm16(cnt, lane ^ s)
    return cnt


def _body(idx_hbm, table_hbm, out_hbm, idx_v, buf0, buf1, out_v, t0_v,
          sem0, sem1):
    wid = lax.axis_index("s") * NC + lax.axis_index("c")
    pltpu.sync_copy(idx_hbm.at[wid], idx_v.at[pl.ds(0, IPW)])

    def _remap(k, carry):
        o = pl.multiple_of(k * 16, 8)
        v = idx_v[pl.ds(o, 16)]
        idx_v[pl.ds(o, 16)] = ((v & (-4096)) + ((v & 1023) << 2)
                               + ((v >> 10) & 3))
        return carry

    lax.fori_loop(0, IPW // 16, _remap, 0, unroll=8)
    pltpu.sync_copy(table_hbm.at[pl.ds(0, 1)], t0_v)
    t0a = t0_v[0, pl.ds(0, 16)]
    t0b = t0_v[0, pl.ds(16, 16)]
    bufs = (buf0, buf1)
    sems = (sem0, sem1)

    for b in (0, 1):
        _issue(table_hbm, idx_v, bufs[b], sems[b],
               pl.multiple_of(b * CIDX, 8))

    def outer(g, carry):
        for b in (0, 1):
            cc = g * 2 + b
            buf, sem = bufs[b], sems[b]
            cbase = pl.multiple_of(cc * CIDX, 8)
            _drain(table_hbm, idx_v, buf, sem, cbase)
            for r in range(CROWS):
                row = cc * CROWS + r
                a0, a1 = _accum_row(buf, r * H)
                nnz = _count_nnz(idx_v, pl.multiple_of(row * H, 8))
                nnzf = nnz.astype(jnp.float32)
                n0f = jnp.float32(H) - nnzf
                inv = 1.0 / nnzf
                out_v[row, pl.ds(0, 16)] = (a0 - n0f * t0a) * inv
                out_v[row, pl.ds(16, 16)] = (a1 - n0f * t0b) * inv
            nxt = cc + 2

            @pl.when(nxt < NCH)
            def _():
                _issue(table_hbm, idx_v, buf, sem,
                       pl.multiple_of(nxt * CIDX, 8))
        return carry

    lax.fori_loop(0, NCH // 2, outer, 0)
    base = pl.multiple_of(wid * BPW, 8)
    pltpu.sync_copy(out_v, out_hbm.at[pl.ds(base, BPW)])


_doc2vec_sc = functools.partial(
    pl.kernel,
    mesh=plsc.VectorSubcoreMesh(core_axis_name="c", subcore_axis_name="s"),
    compiler_params=pltpu.CompilerParams(use_tc_tiling_on_sc=False),
    out_type=jax.ShapeDtypeStruct((B, D), jnp.float32),
    scratch_types=[
        pltpu.VMEM((IPW + 16,), jnp.int32),
        pltpu.VMEM((CIDX, D), jnp.float32),
        pltpu.VMEM((CIDX, D), jnp.float32),
        pltpu.VMEM((BPW, D), jnp.float32),
        pltpu.VMEM((1, D), jnp.float32),
        pltpu.SemaphoreType.DMA,
        pltpu.SemaphoreType.DMA,
    ],
)(_body)


_TCHUNK = 1024
_TQ = 128 // D
_TGRID = -(-VOCAB // (_TCHUNK * _TQ))
_TROWS = _TGRID * _TCHUNK


def _transpose_body(src_ref, dst_ref):
    v = jnp.concatenate(
        [src_ref[:, _TCHUNK * q:_TCHUNK * (q + 1)] for q in range(_TQ)],
        axis=0)
    dst_ref[...] = v.T


_transpose_tc = pl.pallas_call(
    _transpose_body,
    grid=(_TGRID,),
    in_specs=[pl.BlockSpec((D, _TCHUNK * _TQ), lambda i: (0, i))],
    out_specs=pl.BlockSpec((_TCHUNK, 128), lambda i: (i, 0)),
    out_shape=jax.ShapeDtypeStruct((_TROWS, 128), jnp.float32),
)


@jax.jit
def kernel(word_indices, table):
    idx = word_indices.reshape(NW, IPW)
    table_pk = _transpose_tc(jnp.swapaxes(table, 0, 1))
    return _doc2vec_sc(idx, table_pk.reshape(_TROWS * _TQ, D))


# speedup vs baseline: 4.0434x; 1.2648x over previous
import functools

import jax
import jax.numpy as jnp
from jax import lax
from jax.experimental import pallas as pl
from jax.experimental.pallas import tpu as pltpu
from jax.experimental.pallas import tpu_sc as plsc

VOCAB = 1_000_000
D = 32
B = 4096
H = 200

NC = 2
NS = 16
NW = NC * NS
BPW = B // NW
IPW = BPW * H
CROWS = 4
CIDX = CROWS * H
NCH = BPW // CROWS
SLICES = [(o, min(128, CIDX - o)) for o in range(0, CIDX, 128)]


def _copies(table_hbm, idx_v, buf, sem, cbase):
    out = []
    for (o, n) in SLICES:
        src = table_hbm.at[idx_v.at[pl.ds(cbase + o, n)]]
        dst = buf.at[pl.ds(o, n)]
        out.append(pltpu.make_async_copy(src, dst, sem))
    return out


def _issue(table_hbm, idx_v, buf, sem, cbase):
    for c in _copies(table_hbm, idx_v, buf, sem, cbase):
        c.start()


def _drain(table_hbm, idx_v, buf, sem, cbase):
    for c in _copies(table_hbm, idx_v, buf, sem, cbase):
        c.wait()


def _accum_row(buf, rbase):
    def body(j, accs):
        a0, a1 = accs
        a0 = a0 + buf[rbase + j, pl.ds(0, 16)]
        a1 = a1 + buf[rbase + j, pl.ds(16, 16)]
        return (a0, a1)

    z = jnp.zeros((16,), jnp.float32)
    return lax.fori_loop(0, H, body, (z, z), unroll=8)


_GATHER_DNUMS = lax.GatherDimensionNumbers(
    offset_dims=(), collapsed_slice_dims=(0,), start_index_map=(0,))


def _perm16(x, perm):
    return lax.gather(x, perm[:, None], _GATHER_DNUMS, (1,),
                      mode=lax.GatherScatterMode.PROMISE_IN_BOUNDS)


def _count_nnz(idx_v, ibase):
    lane = lax.iota(jnp.int32, 16)
    cnt = jnp.zeros((16,), jnp.int32)
    one = jnp.ones((16,), jnp.int32)
    zero = jnp.zeros((16,), jnp.int32)
    for k in range(H // 16 + 1):
        v = idx_v[pl.ds(ibase + 16 * k, 16)]
        ok = v != 0
        if k == H // 16:
            ok = jnp.logical_and(ok, lane < H - 16 * k)
        cnt = cnt + jnp.where(ok, one, zero)
    for s in (1, 2, 4, 8):
        cnt = cnt + _perm16(cnt, lane ^ s)
    return cnt


def _body(idx_hbm, table_hbm, out_hbm, idx_v, buf0, buf1, out_v, t0_v,
          sem0, sem1):
    wid = lax.axis_index("s") * NC + lax.axis_index("c")
    pltpu.sync_copy(idx_hbm.at[wid], idx_v.at[pl.ds(0, IPW)])

    def _remap(k, carry):
        o = pl.multiple_of(k * 16, 8)
        v = idx_v[pl.ds(o, 16)]
        idx_v[pl.ds(o, 16)] = ((v & (-4 * _TCHUNK))
                               + ((v & (_TCHUNK - 1)) << 2)
                               + ((v >> _TSHIFT) & 3))
        return carry

    lax.fori_loop(0, IPW // 16, _remap, 0, unroll=8)
    pltpu.sync_copy(table_hbm.at[pl.ds(0, 1)], t0_v)
    t0a = t0_v[0, pl.ds(0, 16)]
    t0b = t0_v[0, pl.ds(16, 16)]
    bufs = (buf0, buf1)
    sems = (sem0, sem1)

    for b in (0, 1):
        _issue(table_hbm, idx_v, bufs[b], sems[b],
               pl.multiple_of(b * CIDX, 8))

    def outer(g, carry):
        for b in (0, 1):
            cc = g * 2 + b
            buf, sem = bufs[b], sems[b]
            cbase = pl.multiple_of(cc * CIDX, 8)
            _drain(table_hbm, idx_v, buf, sem, cbase)
            for r in range(CROWS):
                row = cc * CROWS + r
                a0, a1 = _accum_row(buf, r * H)
                nnz = _count_nnz(idx_v, pl.multiple_of(row * H, 8))
                nnzf = nnz.astype(jnp.float32)
                n0f = jnp.float32(H) - nnzf
                inv = 1.0 / nnzf
                out_v[row, pl.ds(0, 16)] = (a0 - n0f * t0a) * inv
                out_v[row, pl.ds(16, 16)] = (a1 - n0f * t0b) * inv
            nxt = cc + 2

            @pl.when(nxt < NCH)
            def _():
                _issue(table_hbm, idx_v, buf, sem,
                       pl.multiple_of(nxt * CIDX, 8))
        return carry

    lax.fori_loop(0, NCH // 2, outer, 0)
    base = pl.multiple_of(wid * BPW, 8)
    pltpu.sync_copy(out_v, out_hbm.at[pl.ds(base, BPW)])


_doc2vec_sc = functools.partial(
    pl.kernel,
    mesh=plsc.VectorSubcoreMesh(core_axis_name="c", subcore_axis_name="s"),
    compiler_params=pltpu.CompilerParams(use_tc_tiling_on_sc=False),
    out_type=jax.ShapeDtypeStruct((B, D), jnp.float32),
    scratch_types=[
        pltpu.VMEM((IPW + 16,), jnp.int32),
        pltpu.VMEM((CIDX, D), jnp.float32),
        pltpu.VMEM((CIDX, D), jnp.float32),
        pltpu.VMEM((BPW, D), jnp.float32),
        pltpu.VMEM((1, D), jnp.float32),
        pltpu.SemaphoreType.DMA,
        pltpu.SemaphoreType.DMA,
    ],
)(_body)


_TCHUNK = 2048
_TSHIFT = _TCHUNK.bit_length() - 1
_TQ = 128 // D
_TGRID = -(-VOCAB // (_TCHUNK * _TQ))
_TROWS = _TGRID * _TCHUNK


def _transpose_body(src_ref, dst_ref):
    v = jnp.concatenate(
        [src_ref[:, _TCHUNK * q:_TCHUNK * (q + 1)] for q in range(_TQ)],
        axis=0)
    dst_ref[...] = v.T


_transpose_tc = pl.pallas_call(
    _transpose_body,
    grid=(_TGRID,),
    in_specs=[pl.BlockSpec((D, _TCHUNK * _TQ), lambda i: (0, i))],
    out_specs=pl.BlockSpec((_TCHUNK, 128), lambda i: (i, 0)),
    out_shape=jax.ShapeDtypeStruct((_TROWS, 128), jnp.float32),
)


@jax.jit
def kernel(word_indices, table):
    idx = word_indices.reshape(NW, IPW)
    table_pk = _transpose_tc(jnp.swapaxes(table, 0, 1))
    return _doc2vec_sc(idx, table_pk.reshape(_TROWS * _TQ, D))


# speedup vs baseline: 4.8541x; 1.2005x over previous
import functools

import jax
import jax.numpy as jnp
from jax import lax
from jax.experimental import pallas as pl
from jax.experimental.pallas import tpu as pltpu
from jax.experimental.pallas import tpu_sc as plsc

VOCAB = 1_000_000
D = 32
B = 4096
H = 200

NC = 2
NS = 16
NW = NC * NS
BPW = B // NW
IPW = BPW * H
CROWS = 4
CIDX = CROWS * H
NCH = BPW // CROWS
SLICES = [(o, min(128, CIDX - o)) for o in range(0, CIDX, 128)]


def _copies(table_hbm, idx_v, buf, sem, cbase):
    out = []
    for (o, n) in SLICES:
        src = table_hbm.at[idx_v.at[pl.ds(cbase + o, n)]]
        dst = buf.at[pl.ds(o, n)]
        out.append(pltpu.make_async_copy(src, dst, sem))
    return out


def _issue(table_hbm, idx_v, buf, sem, cbase):
    for c in _copies(table_hbm, idx_v, buf, sem, cbase):
        c.start()


def _drain(table_hbm, idx_v, buf, sem, cbase):
    for c in _copies(table_hbm, idx_v, buf, sem, cbase):
        c.wait()


def _accum_row(buf, rbase):
    def body(j, accs):
        a0, a1 = accs
        a0 = a0 + buf[rbase + j, pl.ds(0, 16)]
        a1 = a1 + buf[rbase + j, pl.ds(16, 16)]
        return (a0, a1)

    z = jnp.zeros((16,), jnp.float32)
    return lax.fori_loop(0, H, body, (z, z), unroll=8)


_GATHER_DNUMS = lax.GatherDimensionNumbers(
    offset_dims=(), collapsed_slice_dims=(0,), start_index_map=(0,))


def _perm16(x, perm):
    return lax.gather(x, perm[:, None], _GATHER_DNUMS, (1,),
                      mode=lax.GatherScatterMode.PROMISE_IN_BOUNDS)


def _count_nnz(idx_v, ibase):
    lane = lax.iota(jnp.int32, 16)
    cnt = jnp.zeros((16,), jnp.int32)
    one = jnp.ones((16,), jnp.int32)
    zero = jnp.zeros((16,), jnp.int32)
    for k in range(H // 16 + 1):
        v = idx_v[pl.ds(ibase + 16 * k, 16)]
        ok = v != 0
        if k == H // 16:
            ok = jnp.logical_and(ok, lane < H - 16 * k)
        cnt = cnt + jnp.where(ok, one, zero)
    for s in (1, 2, 4, 8):
        cnt = cnt + _perm16(cnt, lane ^ s)
    return cnt


def _body(idx_hbm, table_hbm, out_hbm, idx_v, buf0, buf1, out_v, t0_v,
          sem0, sem1):
    wid = lax.axis_index("s") * NC + lax.axis_index("c")
    pltpu.sync_copy(idx_hbm.at[wid], idx_v.at[pl.ds(0, IPW)])

    def _remap(k, carry):
        o = pl.multiple_of(k * 16, 8)
        v = idx_v[pl.ds(o, 16)]
        idx_v[pl.ds(o, 16)] = ((v & (-4 * _TCHUNK))
                               + ((v & (_TCHUNK - 1)) << 2)
                               + ((v >> _TSHIFT) & 3))
        return carry

    lax.fori_loop(0, IPW // 16, _remap, 0, unroll=8)
    pltpu.sync_copy(table_hbm.at[pl.ds(0, 1)], t0_v)
    t0a = t0_v[0, pl.ds(0, 16)]
    t0b = t0_v[0, pl.ds(16, 16)]
    bufs = (buf0, buf1)
    sems = (sem0, sem1)

    for b in (0, 1):
        _issue(table_hbm, idx_v, bufs[b], sems[b],
               pl.multiple_of(b * CIDX, 8))

    def outer(g, carry):
        for b in (0, 1):
            cc = g * 2 + b
            buf, sem = bufs[b], sems[b]
            cbase = pl.multiple_of(cc * CIDX, 8)
            _drain(table_hbm, idx_v, buf, sem, cbase)
            for r in range(CROWS):
                row = cc * CROWS + r
                a0, a1 = _accum_row(buf, r * H)
                nnz = _count_nnz(idx_v, pl.multiple_of(row * H, 8))
                nnzf = nnz.astype(jnp.float32)
                n0f = jnp.float32(H) - nnzf
                inv = 1.0 / nnzf
                out_v[row, pl.ds(0, 16)] = (a0 - n0f * t0a) * inv
                out_v[row, pl.ds(16, 16)] = (a1 - n0f * t0b) * inv
            nxt = cc + 2

            @pl.when(nxt < NCH)
            def _():
                _issue(table_hbm, idx_v, buf, sem,
                       pl.multiple_of(nxt * CIDX, 8))
        return carry

    lax.fori_loop(0, NCH // 2, outer, 0)
    base = pl.multiple_of(wid * BPW, 8)
    pltpu.sync_copy(out_v, out_hbm.at[pl.ds(base, BPW)])


_doc2vec_sc = functools.partial(
    pl.kernel,
    mesh=plsc.VectorSubcoreMesh(core_axis_name="c", subcore_axis_name="s"),
    compiler_params=pltpu.CompilerParams(use_tc_tiling_on_sc=False),
    out_type=jax.ShapeDtypeStruct((B, D), jnp.float32),
    scratch_types=[
        pltpu.VMEM((IPW + 16,), jnp.int32),
        pltpu.VMEM((CIDX, D), jnp.float32),
        pltpu.VMEM((CIDX, D), jnp.float32),
        pltpu.VMEM((BPW, D), jnp.float32),
        pltpu.VMEM((1, D), jnp.float32),
        pltpu.SemaphoreType.DMA,
        pltpu.SemaphoreType.DMA,
    ],
)(_body)


_TCHUNK = 4096
_TSHIFT = _TCHUNK.bit_length() - 1
_TQ = 128 // D
_TGRID = -(-VOCAB // (_TCHUNK * _TQ))
_TROWS = _TGRID * _TCHUNK


def _transpose_body(src_ref, dst_ref):
    v = jnp.concatenate(
        [src_ref[:, _TCHUNK * q:_TCHUNK * (q + 1)] for q in range(_TQ)],
        axis=0)
    dst_ref[...] = v.T


_transpose_tc = pl.pallas_call(
    _transpose_body,
    grid=(_TGRID,),
    in_specs=[pl.BlockSpec((D, _TCHUNK * _TQ), lambda i: (0, i))],
    out_specs=pl.BlockSpec((_TCHUNK, 128), lambda i: (i, 0)),
    out_shape=jax.ShapeDtypeStruct((_TROWS, 128), jnp.float32),
)


@jax.jit
def kernel(word_indices, table):
    idx = word_indices.reshape(NW, IPW)
    table_pk = _transpose_tc(jnp.swapaxes(table, 0, 1))
    return _doc2vec_sc(idx, table_pk.reshape(_TROWS * _TQ, D))


# speedup vs baseline: 5.2765x; 1.0870x over previous
import functools

import jax
import jax.numpy as jnp
from jax import lax
from jax.experimental import pallas as pl
from jax.experimental.pallas import tpu as pltpu
from jax.experimental.pallas import tpu_sc as plsc

VOCAB = 1_000_000
D = 32
B = 4096
H = 200

NC = 2
NS = 16
NW = NC * NS
BPW = B // NW
IPW = BPW * H
CROWS = 4
CIDX = CROWS * H
NCH = BPW // CROWS
SLICES = [(o, min(128, CIDX - o)) for o in range(0, CIDX, 128)]


def _copies(table_hbm, idx_v, buf, sem, cbase):
    out = []
    for (o, n) in SLICES:
        src = table_hbm.at[idx_v.at[pl.ds(cbase + o, n)]]
        dst = buf.at[pl.ds(o, n)]
        out.append(pltpu.make_async_copy(src, dst, sem))
    return out


def _issue(table_hbm, idx_v, buf, sem, cbase):
    for c in _copies(table_hbm, idx_v, buf, sem, cbase):
        c.start()


def _drain(table_hbm, idx_v, buf, sem, cbase):
    for c in _copies(table_hbm, idx_v, buf, sem, cbase):
        c.wait()


def _accum_row(buf, rbase):
    def body(j, accs):
        a0, a1 = accs
        a0 = a0 + buf[rbase + j, pl.ds(0, 16)]
        a1 = a1 + buf[rbase + j, pl.ds(16, 16)]
        return (a0, a1)

    z = jnp.zeros((16,), jnp.float32)
    return lax.fori_loop(0, H, body, (z, z), unroll=8)


_GATHER_DNUMS = lax.GatherDimensionNumbers(
    offset_dims=(), collapsed_slice_dims=(0,), start_index_map=(0,))


def _perm16(x, perm):
    return lax.gather(x, perm[:, None], _GATHER_DNUMS, (1,),
                      mode=lax.GatherScatterMode.PROMISE_IN_BOUNDS)


def _count_nnz(idx_v, ibase):
    lane = lax.iota(jnp.int32, 16)
    cnt = jnp.zeros((16,), jnp.int32)
    one = jnp.ones((16,), jnp.int32)
    zero = jnp.zeros((16,), jnp.int32)
    for k in range(H // 16 + 1):
        v = idx_v[pl.ds(ibase + 16 * k, 16)]
        ok = v != 0
        if k == H // 16:
            ok = jnp.logical_and(ok, lane < H - 16 * k)
        cnt = cnt + jnp.where(ok, one, zero)
    for s in (1, 2, 4, 8):
        cnt = cnt + _perm16(cnt, lane ^ s)
    return cnt


def _body(idx_hbm, table_hbm, out_hbm, idx_v, buf0, buf1, out_v, t0_v,
          sem0, sem1):
    wid = lax.axis_index("s") * NC + lax.axis_index("c")
    pltpu.sync_copy(idx_hbm.at[wid], idx_v.at[pl.ds(0, IPW)])

    def _remap(k, carry):
        o = pl.multiple_of(k * 16, 8)
        v = idx_v[pl.ds(o, 16)]
        idx_v[pl.ds(o, 16)] = ((v & (-4 * _TCHUNK))
                               + ((v & (_TCHUNK - 1)) << 2)
                               + ((v >> _TSHIFT) & 3))
        return carry

    lax.fori_loop(0, IPW // 16, _remap, 0, unroll=8)
    pltpu.sync_copy(table_hbm.at[pl.ds(0, 1)], t0_v)
    t0a = t0_v[0, pl.ds(0, 16)]
    t0b = t0_v[0, pl.ds(16, 16)]
    bufs = (buf0, buf1)
    sems = (sem0, sem1)

    for b in (0, 1):
        _issue(table_hbm, idx_v, bufs[b], sems[b],
               pl.multiple_of(b * CIDX, 8))

    def outer(g, carry):
        for b in (0, 1):
            cc = g * 2 + b
            buf, sem = bufs[b], sems[b]
            cbase = pl.multiple_of(cc * CIDX, 8)
            _drain(table_hbm, idx_v, buf, sem, cbase)
            for r in range(CROWS):
                row = cc * CROWS + r
                a0, a1 = _accum_row(buf, r * H)
                nnz = _count_nnz(idx_v, pl.multiple_of(row * H, 8))
                nnzf = nnz.astype(jnp.float32)
                n0f = jnp.float32(H) - nnzf
                inv = 1.0 / nnzf
                out_v[row, pl.ds(0, 16)] = (a0 - n0f * t0a) * inv
                out_v[row, pl.ds(16, 16)] = (a1 - n0f * t0b) * inv
            nxt = cc + 2

            @pl.when(nxt < NCH)
            def _():
                _issue(table_hbm, idx_v, buf, sem,
                       pl.multiple_of(nxt * CIDX, 8))
        return carry

    lax.fori_loop(0, NCH // 2, outer, 0)
    base = pl.multiple_of(wid * BPW, 8)
    pltpu.sync_copy(out_v, out_hbm.at[pl.ds(base, BPW)])


_doc2vec_sc = functools.partial(
    pl.kernel,
    mesh=plsc.VectorSubcoreMesh(core_axis_name="c", subcore_axis_name="s"),
    compiler_params=pltpu.CompilerParams(use_tc_tiling_on_sc=False),
    out_type=jax.ShapeDtypeStruct((B, D), jnp.float32),
    scratch_types=[
        pltpu.VMEM((IPW + 16,), jnp.int32),
        pltpu.VMEM((CIDX, D), jnp.float32),
        pltpu.VMEM((CIDX, D), jnp.float32),
        pltpu.VMEM((BPW, D), jnp.float32),
        pltpu.VMEM((1, D), jnp.float32),
        pltpu.SemaphoreType.DMA,
        pltpu.SemaphoreType.DMA,
    ],
)(_body)


_TCHUNK = 8192
_TSHIFT = _TCHUNK.bit_length() - 1
_TQ = 128 // D
_TGRID = -(-VOCAB // (_TCHUNK * _TQ))
_TROWS = _TGRID * _TCHUNK


def _transpose_body(src_ref, dst_ref):
    v = jnp.concatenate(
        [src_ref[:, _TCHUNK * q:_TCHUNK * (q + 1)] for q in range(_TQ)],
        axis=0)
    dst_ref[...] = v.T


_transpose_tc = pl.pallas_call(
    _transpose_body,
    grid=(_TGRID,),
    in_specs=[pl.BlockSpec((D, _TCHUNK * _TQ), lambda i: (0, i))],
    out_specs=pl.BlockSpec((_TCHUNK, 128), lambda i: (i, 0)),
    out_shape=jax.ShapeDtypeStruct((_TROWS, 128), jnp.float32),
)


@jax.jit
def kernel(word_indices, table):
    idx = word_indices.reshape(NW, IPW)
    table_pk = _transpose_tc(jnp.swapaxes(table, 0, 1))
    return _doc2vec_sc(idx, table_pk.reshape(_TROWS * _TQ, D))


# speedup vs baseline: 5.2928x; 1.0031x over previous
import functools

import jax
import jax.numpy as jnp
from jax import lax
from jax.experimental import pallas as pl
from jax.experimental.pallas import tpu as pltpu
from jax.experimental.pallas import tpu_sc as plsc

VOCAB = 1_000_000
D = 32
B = 4096
H = 200

NC = 2
NS = 16
NW = NC * NS
BPW = B // NW
IPW = BPW * H
CROWS = 4
CIDX = CROWS * H
NCH = BPW // CROWS


def _accum_row(buf, rbase):
    def body(j, accs):
        a0, a1 = accs
        a0 = a0 + buf[rbase + j, pl.ds(0, 16)]
        a1 = a1 + buf[rbase + j, pl.ds(16, 16)]
        return (a0, a1)

    z = jnp.zeros((16,), jnp.float32)
    return lax.fori_loop(0, H, body, (z, z), unroll=8)


_GATHER_DNUMS = lax.GatherDimensionNumbers(
    offset_dims=(), collapsed_slice_dims=(0,), start_index_map=(0,))


def _perm16(x, perm):
    return lax.gather(x, perm[:, None], _GATHER_DNUMS, (1,),
                      mode=lax.GatherScatterMode.PROMISE_IN_BOUNDS)


ROW_SLICES = ((0, 104), (104, 96))


def _fmap(v):
    return ((v & (-4 * _TCHUNK)) + ((v & (_TCHUNK - 1)) << 2)
            + ((v >> _TSHIFT) & 3))


def _copies(table_hbm, idx_v, buf, sem, cc):
    out = []
    for r in range(CROWS):
        row = cc * CROWS + r
        for (o, n) in ROW_SLICES:
            src = table_hbm.at[idx_v.at[row, pl.ds(o, n)]]
            dst = buf.at[pl.ds(r * H + o, n)]
            out.append(pltpu.make_async_copy(src, dst, sem))
    return out


def _remap_chunk(idx_v, cc):
    lane = lax.iota(jnp.int32, 16)
    for r in range(CROWS):
        row = cc * CROWS + r
        for k in range(H // 16):
            v = idx_v[row, pl.ds(16 * k, 16)]
            idx_v[row, pl.ds(16 * k, 16)] = _fmap(v)
        v = idx_v[row, pl.ds(H - 16, 16)]
        idx_v[row, pl.ds(H - 16, 16)] = jnp.where(lane >= 8, _fmap(v), v)


def _count_nnz(idx_v, row):
    lane = lax.iota(jnp.int32, 16)
    cnt = jnp.zeros((16,), jnp.int32)
    one = jnp.ones((16,), jnp.int32)
    zero = jnp.zeros((16,), jnp.int32)
    for k in range(H // 16):
        v = idx_v[row, pl.ds(16 * k, 16)]
        cnt = cnt + jnp.where(v != 0, one, zero)
    v = idx_v[row, pl.ds(H - 16, 16)]
    cnt = cnt + jnp.where(jnp.logical_and(v != 0, lane >= 8), one, zero)
    for s in (1, 2, 4, 8):
        cnt = cnt + _perm16(cnt, lane ^ s)
    return cnt


def _body(idx_hbm, table_hbm, out_hbm, idx_v, buf0, buf1, out_v, t0_v,
          sem0, sem1):
    wid = lax.axis_index("s") * NC + lax.axis_index("c")
    pltpu.sync_copy(idx_hbm.at[pl.ds(pl.multiple_of(wid * BPW, 8), BPW)],
                    idx_v)
    pltpu.sync_copy(table_hbm.at[pl.ds(0, 1)], t0_v)
    t0a = t0_v[0, pl.ds(0, 16)]
    t0b = t0_v[0, pl.ds(16, 16)]
    bufs = (buf0, buf1)
    sems = (sem0, sem1)

    for b in (0, 1):
        _remap_chunk(idx_v, b)
        for c in _copies(table_hbm, idx_v, bufs[b], sems[b], b):
            c.start()

    def outer(g, carry):
        for b in (0, 1):
            cc = g * 2 + b
            buf, sem = bufs[b], sems[b]
            for c in _copies(table_hbm, idx_v, buf, sem, cc):
                c.wait()
            for r in range(CROWS):
                row = cc * CROWS + r
                a0, a1 = _accum_row(buf, r * H)
                nnz = _count_nnz(idx_v, row)
                nnzf = nnz.astype(jnp.float32)
                n0f = jnp.float32(H) - nnzf
                inv = 1.0 / nnzf
                out_v[row, pl.ds(0, 16)] = (a0 - n0f * t0a) * inv
                out_v[row, pl.ds(16, 16)] = (a1 - n0f * t0b) * inv
            nxt = cc + 2

            @pl.when(nxt < NCH)
            def _():
                _remap_chunk(idx_v, nxt)
                for c in _copies(table_hbm, idx_v, buf, sem, nxt):
                    c.start()

        return carry

    lax.fori_loop(0, NCH // 2, outer, 0)
    base = pl.multiple_of(wid * BPW, 8)
    pltpu.sync_copy(out_v, out_hbm.at[pl.ds(base, BPW)])


_doc2vec_sc = functools.partial(
    pl.kernel,
    mesh=plsc.VectorSubcoreMesh(core_axis_name="c", subcore_axis_name="s"),
    compiler_params=pltpu.CompilerParams(use_tc_tiling_on_sc=False),
    out_type=jax.ShapeDtypeStruct((B, D), jnp.float32),
    scratch_types=[
        pltpu.VMEM((BPW, H), jnp.int32),
        pltpu.VMEM((CIDX, D), jnp.float32),
        pltpu.VMEM((CIDX, D), jnp.float32),
        pltpu.VMEM((BPW, D), jnp.float32),
        pltpu.VMEM((1, D), jnp.float32),
        pltpu.SemaphoreType.DMA,
        pltpu.SemaphoreType.DMA,
    ],
)(_body)


_TCHUNK = 8192
_TSHIFT = _TCHUNK.bit_length() - 1
_TQ = 128 // D
_TGRID = -(-VOCAB // (_TCHUNK * _TQ))
_TROWS = _TGRID * _TCHUNK


def _transpose_body(src_ref, dst_ref):
    v = jnp.concatenate(
        [src_ref[:, _TCHUNK * q:_TCHUNK * (q + 1)] for q in range(_TQ)],
        axis=0)
    dst_ref[...] = v.T


_transpose_tc = pl.pallas_call(
    _transpose_body,
    grid=(_TGRID,),
    in_specs=[pl.BlockSpec((D, _TCHUNK * _TQ), lambda i: (0, i))],
    out_specs=pl.BlockSpec((_TCHUNK, 128), lambda i: (i, 0)),
    out_shape=jax.ShapeDtypeStruct((_TROWS, 128), jnp.float32),
)


@jax.jit
def kernel(word_indices, table):
    table_pk = _transpose_tc(jnp.swapaxes(table, 0, 1))
    return _doc2vec_sc(word_indices, table_pk.reshape(_TROWS * _TQ, D))


# speedup vs baseline: 5.3375x; 1.0084x over previous
import functools

import jax
import jax.numpy as jnp
from jax import lax
from jax.experimental import pallas as pl
from jax.experimental.pallas import tpu as pltpu
from jax.experimental.pallas import tpu_sc as plsc

VOCAB = 1_000_000
D = 32
B = 4096
H = 200

NC = 2
NS = 16
NW = NC * NS
BPW = B // NW
IPW = BPW * H
CROWS = 4
CIDX = CROWS * H
NCH = BPW // CROWS


def _accum_row(buf, rbase):
    def body(j, accs):
        a0, a1 = accs
        a0 = a0 + buf[rbase + j, pl.ds(0, 16)]
        a1 = a1 + buf[rbase + j, pl.ds(16, 16)]
        return (a0, a1)

    z = jnp.zeros((16,), jnp.float32)
    return lax.fori_loop(0, H, body, (z, z), unroll=8)


_GATHER_DNUMS = lax.GatherDimensionNumbers(
    offset_dims=(), collapsed_slice_dims=(0,), start_index_map=(0,))


def _perm16(x, perm):
    return lax.gather(x, perm[:, None], _GATHER_DNUMS, (1,),
                      mode=lax.GatherScatterMode.PROMISE_IN_BOUNDS)


ROW_SLICES = ((0, 104), (104, 96))


def _fmap(v):
    return ((v & (-4 * _TCHUNK)) + ((v & (_TCHUNK - 1)) << 2)
            + ((v >> _TSHIFT) & 3))


def _copies(table_hbm, idx_v, buf, sem, cc):
    out = []
    for r in range(CROWS):
        row = cc * CROWS + r
        for (o, n) in ROW_SLICES:
            src = table_hbm.at[idx_v.at[row, pl.ds(o, n)]]
            dst = buf.at[pl.ds(r * H + o, n)]
            out.append(pltpu.make_async_copy(src, dst, sem))
    return out


def _remap_chunk(idx_v, cc):
    lane = lax.iota(jnp.int32, 16)
    for r in range(CROWS):
        row = cc * CROWS + r
        for k in range(H // 16):
            v = idx_v[row, pl.ds(16 * k, 16)]
            idx_v[row, pl.ds(16 * k, 16)] = _fmap(v)
        v = idx_v[row, pl.ds(H - 16, 16)]
        idx_v[row, pl.ds(H - 16, 16)] = jnp.where(lane >= 8, _fmap(v), v)


def _count_nnz(idx_v, row):
    lane = lax.iota(jnp.int32, 16)
    cnt = jnp.zeros((16,), jnp.int32)
    one = jnp.ones((16,), jnp.int32)
    zero = jnp.zeros((16,), jnp.int32)
    for k in range(H // 16):
        v = idx_v[row, pl.ds(16 * k, 16)]
        cnt = cnt + jnp.where(v != 0, one, zero)
    v = idx_v[row, pl.ds(H - 16, 16)]
    cnt = cnt + jnp.where(jnp.logical_and(v != 0, lane >= 8), one, zero)
    for s in (1, 2, 4, 8):
        cnt = cnt + _perm16(cnt, lane ^ s)
    return cnt


def _body(idx_hbm, table_hbm, out_hbm, idx_v, buf0, buf1, out_v, t0_v,
          sem0, sem1):
    wid = lax.axis_index("s") * NC + lax.axis_index("c")
    pltpu.sync_copy(idx_hbm.at[pl.ds(pl.multiple_of(wid * BPW, 8), BPW)],
                    idx_v)
    pltpu.sync_copy(table_hbm.at[pl.ds(0, 1)], t0_v)
    t0a = t0_v[0, pl.ds(0, 16)]
    t0b = t0_v[0, pl.ds(16, 16)]
    bufs = (buf0, buf1)
    sems = (sem0, sem1)

    for b in (0, 1):
        _remap_chunk(idx_v, b)
        for c in _copies(table_hbm, idx_v, bufs[b], sems[b], b):
            c.start()

    def outer(g, carry):
        for b in (0, 1):
            cc = g * 2 + b
            buf, sem = bufs[b], sems[b]
            for c in _copies(table_hbm, idx_v, buf, sem, cc):
                c.wait()
            for r in range(CROWS):
                row = cc * CROWS + r
                a0, a1 = _accum_row(buf, r * H)
                nnz = _count_nnz(idx_v, row)
                nnzf = nnz.astype(jnp.float32)
                n0f = jnp.float32(H) - nnzf
                inv = 1.0 / nnzf
                out_v[row, pl.ds(0, 16)] = (a0 - n0f * t0a) * inv
                out_v[row, pl.ds(16, 16)] = (a1 - n0f * t0b) * inv
            nxt = cc + 2

            @pl.when(nxt < NCH)
            def _():
                _remap_chunk(idx_v, nxt)
                for c in _copies(table_hbm, idx_v, buf, sem, nxt):
                    c.start()

        return carry

    lax.fori_loop(0, NCH // 2, outer, 0)
    base = pl.multiple_of(wid * BPW, 8)
    pltpu.sync_copy(out_v, out_hbm.at[pl.ds(base, BPW)])


_doc2vec_sc = functools.partial(
    pl.kernel,
    mesh=plsc.VectorSubcoreMesh(core_axis_name="c", subcore_axis_name="s"),
    compiler_params=pltpu.CompilerParams(use_tc_tiling_on_sc=False),
    out_type=jax.ShapeDtypeStruct((B, D), jnp.float32),
    scratch_types=[
        pltpu.VMEM((BPW, H), jnp.int32),
        pltpu.VMEM((CIDX, D), jnp.float32),
        pltpu.VMEM((CIDX, D), jnp.float32),
        pltpu.VMEM((BPW, D), jnp.float32),
        pltpu.VMEM((1, D), jnp.float32),
        pltpu.SemaphoreType.DMA,
        pltpu.SemaphoreType.DMA,
    ],
)(_body)


_TCHUNK = 16384
_TSHIFT = _TCHUNK.bit_length() - 1
_TQ = 128 // D
_TGRID = -(-VOCAB // (_TCHUNK * _TQ))
_TROWS = _TGRID * _TCHUNK


def _transpose_body(src_ref, dst_ref):
    v = jnp.concatenate(
        [src_ref[:, _TCHUNK * q:_TCHUNK * (q + 1)] for q in range(_TQ)],
        axis=0)
    dst_ref[...] = v.T


_transpose_tc = pl.pallas_call(
    _transpose_body,
    grid=(_TGRID,),
    in_specs=[pl.BlockSpec((D, _TCHUNK * _TQ), lambda i: (0, i))],
    out_specs=pl.BlockSpec((_TCHUNK, 128), lambda i: (i, 0)),
    out_shape=jax.ShapeDtypeStruct((_TROWS, 128), jnp.float32),
)


@jax.jit
def kernel(word_indices, table):
    table_pk = _transpose_tc(jnp.swapaxes(table, 0, 1))
    return _doc2vec_sc(word_indices, table_pk.reshape(_TROWS * _TQ, D))


# speedup vs baseline: 5.3511x; 1.0025x over previous
import functools

import jax
import jax.numpy as jnp
from jax import lax
from jax.experimental import pallas as pl
from jax.experimental.pallas import tpu as pltpu
from jax.experimental.pallas import tpu_sc as plsc

VOCAB = 1_000_000
D = 32
B = 4096
H = 200

NC = 2
NS = 16
NW = NC * NS
BPW = B // NW
CROWS = 4
CIDX = CROWS * H
NCH = BPW // CROWS


def _accum_row(buf, rbase):
    def body(j, accs):
        a0, a1 = accs
        a0 = a0 + buf[rbase + j, pl.ds(0, 16)]
        a1 = a1 + buf[rbase + j, pl.ds(16, 16)]
        return (a0, a1)

    z = jnp.zeros((16,), jnp.float32)
    return lax.fori_loop(0, H, body, (z, z), unroll=8)


_GATHER_DNUMS = lax.GatherDimensionNumbers(
    offset_dims=(), collapsed_slice_dims=(0,), start_index_map=(0,))


def _perm16(x, perm):
    return lax.gather(x, perm[:, None], _GATHER_DNUMS, (1,),
                      mode=lax.GatherScatterMode.PROMISE_IN_BOUNDS)


ROW_SLICES = ((0, 104), (104, 96))


def _fmap(v):
    return ((v & (-4 * _TCHUNK)) + ((v & (_TCHUNK - 1)) << 2)
            + ((v >> _TSHIFT) & 3))


def _copies(table_hbm, idx_v, buf, sem, cc):
    out = []
    for r in range(CROWS):
        row = cc * CROWS + r
        for (o, n) in ROW_SLICES:
            src = table_hbm.at[idx_v.at[row, pl.ds(o, n)]]
            dst = buf.at[pl.ds(r * H + o, n)]
            out.append(pltpu.make_async_copy(src, dst, sem))
    return out


def _remap_chunk(idx_v, cc):
    lane = lax.iota(jnp.int32, 16)
    for r in range(CROWS):
        row = cc * CROWS + r
        for k in range(H // 16):
            v = idx_v[row, pl.ds(16 * k, 16)]
            idx_v[row, pl.ds(16 * k, 16)] = _fmap(v)
        v = idx_v[row, pl.ds(H - 16, 16)]
        idx_v[row, pl.ds(H - 16, 16)] = jnp.where(lane >= 8, _fmap(v), v)


def _count_nnz(idx_v, row):
    lane = lax.iota(jnp.int32, 16)
    cnt = jnp.zeros((16,), jnp.int32)
    one = jnp.ones((16,), jnp.int32)
    zero = jnp.zeros((16,), jnp.int32)
    for k in range(H // 16):
        v = idx_v[row, pl.ds(16 * k, 16)]
        cnt = cnt + jnp.where(v != 0, one, zero)
    v = idx_v[row, pl.ds(H - 16, 16)]
    cnt = cnt + jnp.where(jnp.logical_and(v != 0, lane >= 8), one, zero)
    for s in (1, 2, 4, 8):
        cnt = cnt + _perm16(cnt, lane ^ s)
    return cnt


def _body(idx_hbm, table_hbm, out_hbm, idx_v, buf0, buf1, out_v, t0_v,
          sem0, sem1):
    wid = lax.axis_index("s") * NC + lax.axis_index("c")
    pltpu.sync_copy(idx_hbm.at[pl.ds(pl.multiple_of(wid * BPW, 8), BPW)],
                    idx_v)
    pltpu.sync_copy(table_hbm.at[pl.ds(0, 1)], t0_v)
    t0a = t0_v[0, pl.ds(0, 16)]
    t0b = t0_v[0, pl.ds(16, 16)]
    bufs = (buf0, buf1)
    sems = (sem0, sem1)

    for b in (0, 1):
        _remap_chunk(idx_v, b)
        for c in _copies(table_hbm, idx_v, bufs[b], sems[b], b):
            c.start()

    def outer(g, carry):
        for b in (0, 1):
            cc = g * 2 + b
            buf, sem = bufs[b], sems[b]
            for c in _copies(table_hbm, idx_v, buf, sem, cc):
                c.wait()
            for r in range(CROWS):
                row = cc * CROWS + r
                a0, a1 = _accum_row(buf, r * H)
                nnz = _count_nnz(idx_v, row)
                nnzf = nnz.astype(jnp.float32)
                n0f = jnp.float32(H) - nnzf
                inv = 1.0 / nnzf
                out_v[row, pl.ds(0, 16)] = (a0 - n0f * t0a) * inv
                out_v[row, pl.ds(16, 16)] = (a1 - n0f * t0b) * inv
            nxt = cc + 2

            @pl.when(nxt < NCH)
            def _():
                _remap_chunk(idx_v, nxt)
                for c in _copies(table_hbm, idx_v, buf, sem, nxt):
                    c.start()

        return carry

    lax.fori_loop(0, NCH // 2, outer, 0)
    base = pl.multiple_of(wid * BPW, 8)
    pltpu.sync_copy(out_v, out_hbm.at[pl.ds(base, BPW)])


_doc2vec_sc = functools.partial(
    pl.kernel,
    mesh=plsc.VectorSubcoreMesh(core_axis_name="c", subcore_axis_name="s"),
    compiler_params=pltpu.CompilerParams(use_tc_tiling_on_sc=False),
    out_type=jax.ShapeDtypeStruct((B, D), jnp.float32),
    scratch_types=[
        pltpu.VMEM((BPW, H), jnp.int32),
        pltpu.VMEM((CIDX, D), jnp.float32),
        pltpu.VMEM((CIDX, D), jnp.float32),
        pltpu.VMEM((BPW, D), jnp.float32),
        pltpu.VMEM((1, D), jnp.float32),
        pltpu.SemaphoreType.DMA,
        pltpu.SemaphoreType.DMA,
    ],
)(_body)


_TCHUNK = 16384
_TSHIFT = _TCHUNK.bit_length() - 1
_TQ = 128 // D
_TGRID = -(-VOCAB // (_TCHUNK * _TQ))
_TROWS = _TGRID * _TCHUNK


def _transpose_body(src_ref, dst_ref):
    v = jnp.concatenate(
        [src_ref[:, _TCHUNK * q:_TCHUNK * (q + 1)] for q in range(_TQ)],
        axis=0)
    dst_ref[...] = v.T


_transpose_tc = pl.pallas_call(
    _transpose_body,
    grid=(_TGRID,),
    in_specs=[pl.BlockSpec((D, _TCHUNK * _TQ), lambda i: (0, i))],
    out_specs=pl.BlockSpec((_TCHUNK, 128), lambda i: (i, 0)),
    out_shape=jax.ShapeDtypeStruct((_TROWS, 128), jnp.float32),
)


@jax.jit
def kernel(word_indices, table):
    table_pk = _transpose_tc(jnp.swapaxes(table, 0, 1))
    return _doc2vec_sc(word_indices, table_pk.reshape(_TROWS * _TQ, D))
